# baseline (speedup 1.0000x reference)
import functools

import jax
import jax.numpy as jnp
from jax import lax
from jax.experimental import pallas as pl
from jax.experimental.pallas import tpu as pltpu
from jax.experimental.pallas import tpu_sc as plsc

_K = 6
_HF = 32
_HX = 16
_HM = 32
_H2 = 16
_GW = _HF + _HX
_GWK = _GW * _K
_H2K = _H2 * _K

_NC = 2
_NS = 16
_NW = _NC * _NS


def _prep_body(sp_fea_ref, sp_xyz_ref, fw1_ref, fb1_ref, xw1_ref, xb1_ref,
               gcat_ref):
    gf = jnp.dot(sp_fea_ref[...], fw1_ref[...].T,
                 preferred_element_type=jnp.float32) + fb1_ref[...]
    gx = jnp.dot(sp_xyz_ref[...], xw1_ref[...].T,
                 preferred_element_type=jnp.float32) + xb1_ref[...]
    gcat_ref[...] = jnp.concatenate([gf, gx], axis=1)


def _sc_gather_body(n_chunks, chunk, rows_w, gcat_hbm, idx_hbm, out_hbm,
                    idxv, rowsv, sem):
    wid = lax.axis_index("s") * _NC + lax.axis_index("c")
    base = wid * rows_w
    for ch in range(n_chunks):
        o = base + ch * chunk
        pltpu.sync_copy(idx_hbm.at[pl.ds(o, chunk)], idxv)
        pltpu.async_copy(gcat_hbm.at[idxv], rowsv, sem).wait()
        pltpu.sync_copy(rowsv, out_hbm.at[pl.ds(o, chunk)])


def _b_body(x_ref, xyz_ref, fw1_ref, mw1_ref, mb1_ref, xw1_ref,
            bcat_ref, bm_ref, statsm_ref):
    x = x_ref[...]
    bf = jnp.dot(x, fw1_ref[...].T, preferred_element_type=jnp.float32)
    bm = jnp.dot(x, mw1_ref[...].T,
                 preferred_element_type=jnp.float32) + mb1_ref[...]
    bx = jnp.dot(xyz_ref[...], xw1_ref[...].T,
                 preferred_element_type=jnp.float32)
    bcat_ref[...] = jnp.concatenate([bf, bx], axis=1)
    bm_ref[...] = bm

    @pl.when(pl.program_id(0) == 0)
    def _():
        statsm_ref[...] = jnp.zeros_like(statsm_ref)

    statsm_ref[0:1, 0:_HM] = statsm_ref[0:1, 0:_HM] + jnp.sum(bm, 0, keepdims=True)
    statsm_ref[1:2, 0:_HM] = statsm_ref[1:2, 0:_HM] + jnp.sum(bm * bm, 0, keepdims=True)


def _s_body(hgr_ref, bcat_ref, stats_ref):
    bcat = bcat_ref[...]
    sh = jnp.zeros((1, _GW), jnp.float32)
    sh2 = jnp.zeros((1, _GW), jnp.float32)
    for k in range(_K):
        hck = hgr_ref[:, k * _GW:(k + 1) * _GW] - bcat
        sh = sh + jnp.sum(hck, axis=0, keepdims=True)
        sh2 = sh2 + jnp.sum(hck * hck, axis=0, keepdims=True)

    @pl.when(pl.program_id(0) == 0)
    def _():
        stats_ref[...] = jnp.zeros_like(stats_ref)

    stats_ref[0:1, 0:_GW] = stats_ref[0:1, 0:_GW] + sh
    stats_ref[1:2, 0:_GW] = stats_ref[1:2, 0:_GW] + sh2


def _c_body(n_pts, stats_ref, statsm_ref, fg1_ref, fbe1_ref, xg1_ref,
            xbe1_ref, mg1_ref, mbe1_ref, coefs_ref):
    nk = float(n_pts * _K)
    nf = float(n_pts)
    coefs_ref[...] = jnp.zeros_like(coefs_ref)

    g48 = jnp.concatenate([fg1_ref[...], xg1_ref[...]], axis=1)
    be48 = jnp.concatenate([fbe1_ref[...], xbe1_ref[...]], axis=1)
    mean_h = stats_ref[0:1, 0:_GW] / nk
    var_h = stats_ref[1:2, 0:_GW] / nk - mean_h * mean_h
    a48 = g48 * jax.lax.rsqrt(var_h + 1e-5)
    c48 = be48 - a48 * mean_h
    for k in range(_K):
        coefs_ref[0:1, k * _GW:(k + 1) * _GW] = a48
        coefs_ref[1:2, k * _GW:(k + 1) * _GW] = c48

    mean_m = statsm_ref[0:1, 0:_HM] / nf
    var_m = statsm_ref[1:2, 0:_HM] / nf - mean_m * mean_m
    a_m = mg1_ref[...] * jax.lax.rsqrt(var_m + 1e-5)
    coefs_ref[2:3, 0:_HM] = a_m
    coefs_ref[3:4, 0:_HM] = mbe1_ref[...] - a_m * mean_m


def _d_body(hgr_ref, bcat_ref, bm_ref, idx_ref, x_ref, xyz_ref, coefs_ref,
            it48_ref, wbf_ref, wbx_ref, itile_ref, fb2t_ref, xb2t_ref,
            mw2_ref, mb2_ref, sel_ref,
            out_ref):
    nb = x_ref.shape[0]
    m = out_ref.shape[0]
    a288 = coefs_ref[0:1, :]
    c288 = coefs_ref[1:2, :]
    a_m = coefs_ref[2:3, 0:_HM]
    c_m = coefs_ref[3:4, 0:_HM]

    hm = jnp.maximum(a_m * bm_ref[...] + c_m, 0.0)
    p = jnp.dot(hm, mw2_ref[...].T,
                preferred_element_type=jnp.float32) + mb2_ref[...]
    pp = jnp.sum(p * p, axis=1, keepdims=True)
    pn = jnp.maximum(jnp.sqrt(pp), 1e-12)
    prep6 = jnp.dot(p, itile_ref[...], preferred_element_type=jnp.float32)

    bcatt = jnp.dot(bcat_ref[...], it48_ref[...],
                    preferred_element_type=jnp.float32)
    ht = jnp.maximum((hgr_ref[...] - bcatt) * a288 + c288, 0.0)
    wf = jnp.dot(ht, wbf_ref[...],
                 preferred_element_type=jnp.float32) + fb2t_ref[...]
    wx = jnp.dot(ht, wbx_ref[...],
                 preferred_element_type=jnp.float32) + xb2t_ref[...]

    sel = sel_ref[...]
    u = jnp.dot(wf * prep6, sel, preferred_element_type=jnp.float32)
    v = jnp.dot(wx * prep6, sel, preferred_element_type=jnp.float32)
    s2 = jnp.dot(wf * wf, sel, preferred_element_type=jnp.float32)
    t2 = jnp.dot(wx * wx, sel, preferred_element_type=jnp.float32)

    df = jnp.maximum(jnp.sqrt(s2[:, 0:_K]), 1e-12)
    dx = jnp.maximum(jnp.sqrt(t2[:, 0:_K]), 1e-12)
    logits = (u[:, 0:_K] * v[:, 0:_K]) / (df * dx * (pn * pn))

    mx = jnp.max(logits, axis=1, keepdims=True)
    e = jnp.exp(logits - mx)
    biw = e / jnp.sum(e, axis=1, keepdims=True)

    lane = jax.lax.broadcasted_iota(jnp.int32, (nb, m), 1)
    s = jnp.zeros((nb, m), jnp.float32)
    for k in range(_K):
        s = s + biw[:, k:k + 1] * (idx_ref[:, k:k + 1] == lane).astype(jnp.float32)

    x136 = jnp.concatenate(
        [x_ref[...], xyz_ref[...], jnp.ones((nb, 1), jnp.float32),
         jnp.zeros((nb, 4), jnp.float32)], axis=1)

    dn = (((0,), (0,)), ((), ()))
    upd = jax.lax.dot_general(s, x136, dn, preferred_element_type=jnp.float32)

    @pl.when(pl.program_id(0) == 0)
    def _():
        out_ref[...] = jnp.zeros_like(out_ref)

    out_ref[...] = out_ref[...] + upd

    @pl.when(pl.program_id(0) == pl.num_programs(0) - 1)
    def _():
        den = out_ref[:, 131:132] + 1e-8
        out_ref[...] = out_ref[...] / den


def kernel(sp_fea, sp_xyz, o_p_fea, p_xyz, c2p_idx_abs, c2p_idx, cluster_idx,
           offset, sp_offset,
           fea_w1, fea_b1, fea_g1, fea_be1, fea_w2, fea_b2,
           xyz_w1, xyz_b1, xyz_g1, xyz_be1, xyz_w2, xyz_b2,
           mlp_w1, mlp_b1, mlp_g1, mlp_be1, mlp_w2, mlp_b2):
    n, c = o_p_fea.shape
    m = sp_fea.shape[0]
    f32 = jnp.float32

    nb = n
    for cand in (1000, 500, 250, 200, 100, 50, 25, 10, 8, 5, 4, 2, 1):
        if n % cand == 0:
            nb = cand
            break
    grid = n // nb

    r = lambda v: v.reshape(1, -1)

    zf = jnp.zeros((_GWK, _H2K), f32)
    zx = jnp.zeros((_GWK, _H2K), f32)
    it = jnp.zeros((_H2, _H2K), f32)
    it48 = jnp.zeros((_GW, _GWK), f32)
    sel = jnp.zeros((_H2K, 8), f32)
    eye16 = jnp.eye(_H2, dtype=f32)
    eye48 = jnp.eye(_GW, dtype=f32)
    for k in range(_K):
        zf = zf.at[k * _GW:k * _GW + _HF, k * _H2:(k + 1) * _H2].set(fea_w2.T)
        zx = zx.at[k * _GW + _HF:(k + 1) * _GW, k * _H2:(k + 1) * _H2].set(xyz_w2.T)
        it = it.at[:, k * _H2:(k + 1) * _H2].set(eye16)
        it48 = it48.at[:, k * _GW:(k + 1) * _GW].set(eye48)
        sel = sel.at[k * _H2:(k + 1) * _H2, k].set(1.0)
    fb2t = jnp.tile(fea_b2, _K).reshape(1, _H2K)
    xb2t = jnp.tile(xyz_b2, _K).reshape(1, _H2K)

    gcat = pl.pallas_call(
        _prep_body,
        out_shape=jax.ShapeDtypeStruct((m, _GW), f32),
    )(sp_fea, sp_xyz, fea_w1, r(fea_b1), xyz_w1, r(xyz_b1))

    total = n * _K
    chunk = 960
    n_chunks = -(-(-(-total // _NW)) // chunk)
    rows_w = n_chunks * chunk
    total_pad = rows_w * _NW
    idx_flat = c2p_idx_abs.reshape(-1)
    if total_pad > total:
        idx_flat = jnp.concatenate(
            [idx_flat, jnp.zeros((total_pad - total,), jnp.int32)])

    sc_gather = functools.partial(
        pl.kernel,
        out_type=jax.ShapeDtypeStruct((total_pad, _GW), f32),
        mesh=plsc.VectorSubcoreMesh(core_axis_name="c", subcore_axis_name="s"),
        scratch_types=[
            pltpu.VMEM((chunk,), jnp.int32),
            pltpu.VMEM((chunk, _GW), f32),
            pltpu.SemaphoreType.DMA,
        ],
        compiler_params=pltpu.CompilerParams(use_tc_tiling_on_sc=False),
    )(functools.partial(_sc_gather_body, n_chunks, chunk, rows_w))
    hgr = sc_gather(gcat, idx_flat).reshape(total_pad // _K, _GWK)

    blk = lambda shape: pl.BlockSpec(shape, lambda i: (i, 0))
    full = lambda shape: pl.BlockSpec(shape, lambda i: (0, 0))

    bcat, bm, statsm = pl.pallas_call(
        _b_body,
        grid=(grid,),
        in_specs=[
            blk((nb, c)), blk((nb, 3)),
            full((_HF, c)), full((_HM, c)), full((1, _HM)), full((_HX, 3)),
        ],
        out_specs=[blk((nb, _GW)), blk((nb, _HM)), full((8, 128))],
        out_shape=[
            jax.ShapeDtypeStruct((n, _GW), f32),
            jax.ShapeDtypeStruct((n, _HM), f32),
            jax.ShapeDtypeStruct((8, 128), f32),
        ],
    )(o_p_fea, p_xyz, fea_w1, mlp_w1, r(mlp_b1), xyz_w1)

    stats = pl.pallas_call(
        _s_body,
        grid=(grid,),
        in_specs=[blk((nb, _GWK)), blk((nb, _GW))],
        out_specs=[full((8, 128))],
        out_shape=[jax.ShapeDtypeStruct((8, 128), f32)],
    )(hgr, bcat)[0]

    coefs = pl.pallas_call(
        functools.partial(_c_body, n),
        out_shape=jax.ShapeDtypeStruct((8, _GWK), f32),
    )(stats, statsm, r(fea_g1), r(fea_be1), r(xyz_g1), r(xyz_be1),
      r(mlp_g1), r(mlp_be1))

    out = pl.pallas_call(
        _d_body,
        grid=(grid,),
        in_specs=[
            blk((nb, _GWK)), blk((nb, _GW)), blk((nb, _HM)), blk((nb, _K)),
            blk((nb, c)), blk((nb, 3)),
            full((8, _GWK)), full((_GW, _GWK)),
            full((_GWK, _H2K)), full((_GWK, _H2K)), full((_H2, _H2K)),
            full((1, _H2K)), full((1, _H2K)),
            full((_H2, _HM)), full((1, _H2)), full((_H2K, 8)),
        ],
        out_specs=[full((m, 136))],
        out_shape=[jax.ShapeDtypeStruct((m, 136), f32)],
    )(hgr, bcat, bm, c2p_idx, o_p_fea, p_xyz, coefs,
      it48, zf, zx, it, fb2t, xb2t, mlp_w2, r(mlp_b2), sel)

    out = out[0]
    return (out[:, :c], out[:, c:c + 3])


# speedup vs baseline: 1.0125x; 1.0125x over previous
import functools

import jax
import jax.numpy as jnp
from jax import lax
from jax.experimental import pallas as pl
from jax.experimental.pallas import tpu as pltpu
from jax.experimental.pallas import tpu_sc as plsc

_K = 6
_HF = 32
_HX = 16
_HM = 32
_H2 = 16
_GW = _HF + _HX
_GWK = _GW * _K
_H2K = _H2 * _K

_NC = 2
_NS = 16
_NW = _NC * _NS


def _prep_body(sp_fea_ref, sp_xyz_ref, fw1_ref, fb1_ref, xw1_ref, xb1_ref,
               gcat_ref):
    gf = jnp.dot(sp_fea_ref[...], fw1_ref[...].T,
                 preferred_element_type=jnp.float32) + fb1_ref[...]
    gx = jnp.dot(sp_xyz_ref[...], xw1_ref[...].T,
                 preferred_element_type=jnp.float32) + xb1_ref[...]
    gcat_ref[...] = jnp.concatenate([gf, gx], axis=1)


def _sc_gather_body(n_chunks, chunk, rows_w, gcat_hbm, idx_hbm, out_hbm,
                    idxv0, idxv1, rowsv0, rowsv1, gsem, wsem0, wsem1):
    wid = lax.axis_index("s") * _NC + lax.axis_index("c")
    base = wid * rows_w
    idxv = (idxv0, idxv1)
    rowsv = (rowsv0, rowsv1)
    wsem = (wsem0, wsem1)
    wb = [None, None]
    for ch in range(n_chunks):
        b = ch % 2
        o = base + ch * chunk
        pltpu.sync_copy(idx_hbm.at[pl.ds(o, chunk)], idxv[b])
        if wb[b] is not None:
            wb[b].wait()
            wb[b] = None
        pltpu.async_copy(gcat_hbm.at[idxv[b]], rowsv[b], gsem).wait()
        wb[b] = pltpu.async_copy(rowsv[b], out_hbm.at[pl.ds(o, chunk)],
                                 wsem[b])
    for b in range(2):
        if wb[b] is not None:
            wb[b].wait()


def _b_body(x_ref, xyz_ref, fw1_ref, mw1_ref, mb1_ref, xw1_ref,
            bcat_ref, bm_ref, statsm_ref):
    x = x_ref[...]
    bf = jnp.dot(x, fw1_ref[...].T, preferred_element_type=jnp.float32)
    bm = jnp.dot(x, mw1_ref[...].T,
                 preferred_element_type=jnp.float32) + mb1_ref[...]
    bx = jnp.dot(xyz_ref[...], xw1_ref[...].T,
                 preferred_element_type=jnp.float32)
    bcat_ref[...] = jnp.concatenate([bf, bx], axis=1)
    bm_ref[...] = bm

    @pl.when(pl.program_id(0) == 0)
    def _():
        statsm_ref[...] = jnp.zeros_like(statsm_ref)

    statsm_ref[0:1, 0:_HM] = statsm_ref[0:1, 0:_HM] + jnp.sum(bm, 0, keepdims=True)
    statsm_ref[1:2, 0:_HM] = statsm_ref[1:2, 0:_HM] + jnp.sum(bm * bm, 0, keepdims=True)


def _s_body(hgr_ref, bcat_ref, stats_ref):
    bcat = bcat_ref[...]
    sh = jnp.zeros((1, _GW), jnp.float32)
    sh2 = jnp.zeros((1, _GW), jnp.float32)
    for k in range(_K):
        hck = hgr_ref[:, k * _GW:(k + 1) * _GW] - bcat
        sh = sh + jnp.sum(hck, axis=0, keepdims=True)
        sh2 = sh2 + jnp.sum(hck * hck, axis=0, keepdims=True)

    @pl.when(pl.program_id(0) == 0)
    def _():
        stats_ref[...] = jnp.zeros_like(stats_ref)

    stats_ref[0:1, 0:_GW] = stats_ref[0:1, 0:_GW] + sh
    stats_ref[1:2, 0:_GW] = stats_ref[1:2, 0:_GW] + sh2


def _c_body(n_pts, stats_ref, statsm_ref, fg1_ref, fbe1_ref, xg1_ref,
            xbe1_ref, mg1_ref, mbe1_ref, coefs_ref):
    nk = float(n_pts * _K)
    nf = float(n_pts)
    coefs_ref[...] = jnp.zeros_like(coefs_ref)

    g48 = jnp.concatenate([fg1_ref[...], xg1_ref[...]], axis=1)
    be48 = jnp.concatenate([fbe1_ref[...], xbe1_ref[...]], axis=1)
    mean_h = stats_ref[0:1, 0:_GW] / nk
    var_h = stats_ref[1:2, 0:_GW] / nk - mean_h * mean_h
    a48 = g48 * jax.lax.rsqrt(var_h + 1e-5)
    c48 = be48 - a48 * mean_h
    for k in range(_K):
        coefs_ref[0:1, k * _GW:(k + 1) * _GW] = a48
        coefs_ref[1:2, k * _GW:(k + 1) * _GW] = c48

    mean_m = statsm_ref[0:1, 0:_HM] / nf
    var_m = statsm_ref[1:2, 0:_HM] / nf - mean_m * mean_m
    a_m = mg1_ref[...] * jax.lax.rsqrt(var_m + 1e-5)
    coefs_ref[2:3, 0:_HM] = a_m
    coefs_ref[3:4, 0:_HM] = mbe1_ref[...] - a_m * mean_m


def _d_body(hgr_ref, bcat_ref, bm_ref, idx_ref, x_ref, xyz_ref, coefs_ref,
            it48_ref, wbf_ref, wbx_ref, itile_ref, fb2t_ref, xb2t_ref,
            mw2_ref, mb2_ref, sel_ref,
            out_ref):
    nb = x_ref.shape[0]
    m = out_ref.shape[0]
    a288 = coefs_ref[0:1, :]
    c288 = coefs_ref[1:2, :]
    a_m = coefs_ref[2:3, 0:_HM]
    c_m = coefs_ref[3:4, 0:_HM]

    hm = jnp.maximum(a_m * bm_ref[...] + c_m, 0.0)
    p = jnp.dot(hm, mw2_ref[...].T,
                preferred_element_type=jnp.float32) + mb2_ref[...]
    pp = jnp.sum(p * p, axis=1, keepdims=True)
    pn = jnp.maximum(jnp.sqrt(pp), 1e-12)
    prep6 = jnp.dot(p, itile_ref[...], preferred_element_type=jnp.float32)

    bcatt = jnp.dot(bcat_ref[...], it48_ref[...],
                    preferred_element_type=jnp.float32)
    ht = jnp.maximum((hgr_ref[...] - bcatt) * a288 + c288, 0.0)
    wf = jnp.dot(ht, wbf_ref[...],
                 preferred_element_type=jnp.float32) + fb2t_ref[...]
    wx = jnp.dot(ht, wbx_ref[...],
                 preferred_element_type=jnp.float32) + xb2t_ref[...]

    sel = sel_ref[...]
    u = jnp.dot(wf * prep6, sel, preferred_element_type=jnp.float32)
    v = jnp.dot(wx * prep6, sel, preferred_element_type=jnp.float32)
    s2 = jnp.dot(wf * wf, sel, preferred_element_type=jnp.float32)
    t2 = jnp.dot(wx * wx, sel, preferred_element_type=jnp.float32)

    df = jnp.maximum(jnp.sqrt(s2[:, 0:_K]), 1e-12)
    dx = jnp.maximum(jnp.sqrt(t2[:, 0:_K]), 1e-12)
    logits = (u[:, 0:_K] * v[:, 0:_K]) / (df * dx * (pn * pn))

    mx = jnp.max(logits, axis=1, keepdims=True)
    e = jnp.exp(logits - mx)
    biw = e / jnp.sum(e, axis=1, keepdims=True)

    lane = jax.lax.broadcasted_iota(jnp.int32, (nb, m), 1)
    s = jnp.zeros((nb, m), jnp.float32)
    for k in range(_K):
        s = s + biw[:, k:k + 1] * (idx_ref[:, k:k + 1] == lane).astype(jnp.float32)

    x136 = jnp.concatenate(
        [x_ref[...], xyz_ref[...], jnp.ones((nb, 1), jnp.float32),
         jnp.zeros((nb, 4), jnp.float32)], axis=1)

    dn = (((0,), (0,)), ((), ()))
    upd = jax.lax.dot_general(s, x136, dn, preferred_element_type=jnp.float32)

    @pl.when(pl.program_id(0) == 0)
    def _():
        out_ref[...] = jnp.zeros_like(out_ref)

    out_ref[...] = out_ref[...] + upd

    @pl.when(pl.program_id(0) == pl.num_programs(0) - 1)
    def _():
        den = out_ref[:, 131:132] + 1e-8
        out_ref[...] = out_ref[...] / den


def kernel(sp_fea, sp_xyz, o_p_fea, p_xyz, c2p_idx_abs, c2p_idx, cluster_idx,
           offset, sp_offset,
           fea_w1, fea_b1, fea_g1, fea_be1, fea_w2, fea_b2,
           xyz_w1, xyz_b1, xyz_g1, xyz_be1, xyz_w2, xyz_b2,
           mlp_w1, mlp_b1, mlp_g1, mlp_be1, mlp_w2, mlp_b2):
    n, c = o_p_fea.shape
    m = sp_fea.shape[0]
    f32 = jnp.float32

    nb = n
    for cand in (1000, 500, 250, 200, 100, 50, 25, 10, 8, 5, 4, 2, 1):
        if n % cand == 0:
            nb = cand
            break
    grid = n // nb

    r = lambda v: v.reshape(1, -1)

    zf = jnp.zeros((_GWK, _H2K), f32)
    zx = jnp.zeros((_GWK, _H2K), f32)
    it = jnp.zeros((_H2, _H2K), f32)
    it48 = jnp.zeros((_GW, _GWK), f32)
    sel = jnp.zeros((_H2K, 8), f32)
    eye16 = jnp.eye(_H2, dtype=f32)
    eye48 = jnp.eye(_GW, dtype=f32)
    for k in range(_K):
        zf = zf.at[k * _GW:k * _GW + _HF, k * _H2:(k + 1) * _H2].set(fea_w2.T)
        zx = zx.at[k * _GW + _HF:(k + 1) * _GW, k * _H2:(k + 1) * _H2].set(xyz_w2.T)
        it = it.at[:, k * _H2:(k + 1) * _H2].set(eye16)
        it48 = it48.at[:, k * _GW:(k + 1) * _GW].set(eye48)
        sel = sel.at[k * _H2:(k + 1) * _H2, k].set(1.0)
    fb2t = jnp.tile(fea_b2, _K).reshape(1, _H2K)
    xb2t = jnp.tile(xyz_b2, _K).reshape(1, _H2K)

    gcat = pl.pallas_call(
        _prep_body,
        out_shape=jax.ShapeDtypeStruct((m, _GW), f32),
    )(sp_fea, sp_xyz, fea_w1, r(fea_b1), xyz_w1, r(xyz_b1))

    total = n * _K
    chunk = 960
    n_chunks = -(-(-(-total // _NW)) // chunk)
    rows_w = n_chunks * chunk
    total_pad = rows_w * _NW
    idx_flat = c2p_idx_abs.reshape(-1)
    if total_pad > total:
        idx_flat = jnp.concatenate(
            [idx_flat, jnp.zeros((total_pad - total,), jnp.int32)])

    sc_gather = functools.partial(
        pl.kernel,
        out_type=jax.ShapeDtypeStruct((total_pad, _GW), f32),
        mesh=plsc.VectorSubcoreMesh(core_axis_name="c", subcore_axis_name="s"),
        scratch_types=[
            pltpu.VMEM((chunk,), jnp.int32),
            pltpu.VMEM((chunk,), jnp.int32),
            pltpu.VMEM((chunk, _GW), f32),
            pltpu.VMEM((chunk, _GW), f32),
            pltpu.SemaphoreType.DMA,
            pltpu.SemaphoreType.DMA,
            pltpu.SemaphoreType.DMA,
        ],
        compiler_params=pltpu.CompilerParams(use_tc_tiling_on_sc=False),
    )(functools.partial(_sc_gather_body, n_chunks, chunk, rows_w))
    hgr = sc_gather(gcat, idx_flat).reshape(total_pad // _K, _GWK)

    blk = lambda shape: pl.BlockSpec(shape, lambda i: (i, 0))
    full = lambda shape: pl.BlockSpec(shape, lambda i: (0, 0))

    bcat, bm, statsm = pl.pallas_call(
        _b_body,
        grid=(grid,),
        in_specs=[
            blk((nb, c)), blk((nb, 3)),
            full((_HF, c)), full((_HM, c)), full((1, _HM)), full((_HX, 3)),
        ],
        out_specs=[blk((nb, _GW)), blk((nb, _HM)), full((8, 128))],
        out_shape=[
            jax.ShapeDtypeStruct((n, _GW), f32),
            jax.ShapeDtypeStruct((n, _HM), f32),
            jax.ShapeDtypeStruct((8, 128), f32),
        ],
    )(o_p_fea, p_xyz, fea_w1, mlp_w1, r(mlp_b1), xyz_w1)

    stats = pl.pallas_call(
        _s_body,
        grid=(grid,),
        in_specs=[blk((nb, _GWK)), blk((nb, _GW))],
        out_specs=[full((8, 128))],
        out_shape=[jax.ShapeDtypeStruct((8, 128), f32)],
    )(hgr, bcat)[0]

    coefs = pl.pallas_call(
        functools.partial(_c_body, n),
        out_shape=jax.ShapeDtypeStruct((8, _GWK), f32),
    )(stats, statsm, r(fea_g1), r(fea_be1), r(xyz_g1), r(xyz_be1),
      r(mlp_g1), r(mlp_be1))

    out = pl.pallas_call(
        _d_body,
        grid=(grid,),
        in_specs=[
            blk((nb, _GWK)), blk((nb, _GW)), blk((nb, _HM)), blk((nb, _K)),
            blk((nb, c)), blk((nb, 3)),
            full((8, _GWK)), full((_GW, _GWK)),
            full((_GWK, _H2K)), full((_GWK, _H2K)), full((_H2, _H2K)),
            full((1, _H2K)), full((1, _H2K)),
            full((_H2, _HM)), full((1, _H2)), full((_H2K, 8)),
        ],
        out_specs=[full((m, 136))],
        out_shape=[jax.ShapeDtypeStruct((m, 136), f32)],
    )(hgr, bcat, bm, c2p_idx, o_p_fea, p_xyz, coefs,
      it48, zf, zx, it, fb2t, xb2t, mlp_w2, r(mlp_b2), sel)

    out = out[0]
    return (out[:, :c], out[:, c:c + 3])


# speedup vs baseline: 1.0290x; 1.0163x over previous
import functools

import jax
import jax.numpy as jnp
from jax import lax
from jax.experimental import pallas as pl
from jax.experimental.pallas import tpu as pltpu
from jax.experimental.pallas import tpu_sc as plsc

_K = 6
_HF = 32
_HX = 16
_HM = 32
_H2 = 16
_GW = _HF + _HX
_GWK = _GW * _K
_H2K = _H2 * _K

_NC = 2
_NS = 16
_NW = _NC * _NS


def _prep_body(sp_fea_ref, sp_xyz_ref, fw1_ref, fb1_ref, xw1_ref, xb1_ref,
               gcat_ref):
    gf = jnp.dot(sp_fea_ref[...], fw1_ref[...].T,
                 preferred_element_type=jnp.float32) + fb1_ref[...]
    gx = jnp.dot(sp_xyz_ref[...], xw1_ref[...].T,
                 preferred_element_type=jnp.float32) + xb1_ref[...]
    gcat_ref[...] = jnp.concatenate([gf, gx], axis=1)


_NBUF = 4


def _sc_gather_body(n_chunks, chunk, rows_w, gcat_hbm, idx_hbm, out_hbm,
                    *refs):
    idxv = refs[0:_NBUF]
    rowsv = refs[_NBUF:2 * _NBUF]
    gsem = refs[2 * _NBUF:3 * _NBUF]
    wsem = refs[3 * _NBUF:4 * _NBUF]
    wid = lax.axis_index("s") * _NC + lax.axis_index("c")
    base = wid * rows_w

    g = [None] * _NBUF
    wb = [None] * _NBUF
    for ch in range(min(_NBUF, n_chunks)):
        o = base + ch * chunk
        pltpu.sync_copy(idx_hbm.at[pl.ds(o, chunk)], idxv[ch])
        g[ch] = pltpu.async_copy(gcat_hbm.at[idxv[ch]], rowsv[ch], gsem[ch])
    for ch in range(n_chunks):
        b = ch % _NBUF
        o = base + ch * chunk
        g[b].wait()
        wb[b] = pltpu.async_copy(rowsv[b], out_hbm.at[pl.ds(o, chunk)],
                                 wsem[b])
        nxt = ch + _NBUF
        if nxt < n_chunks:
            o2 = base + nxt * chunk
            wb[b].wait()
            wb[b] = None
            pltpu.sync_copy(idx_hbm.at[pl.ds(o2, chunk)], idxv[b])
            g[b] = pltpu.async_copy(gcat_hbm.at[idxv[b]], rowsv[b], gsem[b])
    for b in range(_NBUF):
        if wb[b] is not None:
            wb[b].wait()


def _b_body(x_ref, xyz_ref, fw1_ref, mw1_ref, mb1_ref, xw1_ref,
            bcat_ref, bm_ref, statsm_ref):
    x = x_ref[...]
    bf = jnp.dot(x, fw1_ref[...].T, preferred_element_type=jnp.float32)
    bm = jnp.dot(x, mw1_ref[...].T,
                 preferred_element_type=jnp.float32) + mb1_ref[...]
    bx = jnp.dot(xyz_ref[...], xw1_ref[...].T,
                 preferred_element_type=jnp.float32)
    bcat_ref[...] = jnp.concatenate([bf, bx], axis=1)
    bm_ref[...] = bm

    @pl.when(pl.program_id(0) == 0)
    def _():
        statsm_ref[...] = jnp.zeros_like(statsm_ref)

    statsm_ref[0:1, 0:_HM] = statsm_ref[0:1, 0:_HM] + jnp.sum(bm, 0, keepdims=True)
    statsm_ref[1:2, 0:_HM] = statsm_ref[1:2, 0:_HM] + jnp.sum(bm * bm, 0, keepdims=True)


def _s_body(hgr_ref, bcat_ref, stats_ref):
    bcat = bcat_ref[...]
    sh = jnp.zeros((1, _GW), jnp.float32)
    sh2 = jnp.zeros((1, _GW), jnp.float32)
    for k in range(_K):
        hck = hgr_ref[:, k * _GW:(k + 1) * _GW] - bcat
        sh = sh + jnp.sum(hck, axis=0, keepdims=True)
        sh2 = sh2 + jnp.sum(hck * hck, axis=0, keepdims=True)

    @pl.when(pl.program_id(0) == 0)
    def _():
        stats_ref[...] = jnp.zeros_like(stats_ref)

    stats_ref[0:1, 0:_GW] = stats_ref[0:1, 0:_GW] + sh
    stats_ref[1:2, 0:_GW] = stats_ref[1:2, 0:_GW] + sh2


def _c_body(n_pts, stats_ref, statsm_ref, fg1_ref, fbe1_ref, xg1_ref,
            xbe1_ref, mg1_ref, mbe1_ref, coefs_ref):
    nk = float(n_pts * _K)
    nf = float(n_pts)
    coefs_ref[...] = jnp.zeros_like(coefs_ref)

    g48 = jnp.concatenate([fg1_ref[...], xg1_ref[...]], axis=1)
    be48 = jnp.concatenate([fbe1_ref[...], xbe1_ref[...]], axis=1)
    mean_h = stats_ref[0:1, 0:_GW] / nk
    var_h = stats_ref[1:2, 0:_GW] / nk - mean_h * mean_h
    a48 = g48 * jax.lax.rsqrt(var_h + 1e-5)
    c48 = be48 - a48 * mean_h
    for k in range(_K):
        coefs_ref[0:1, k * _GW:(k + 1) * _GW] = a48
        coefs_ref[1:2, k * _GW:(k + 1) * _GW] = c48

    mean_m = statsm_ref[0:1, 0:_HM] / nf
    var_m = statsm_ref[1:2, 0:_HM] / nf - mean_m * mean_m
    a_m = mg1_ref[...] * jax.lax.rsqrt(var_m + 1e-5)
    coefs_ref[2:3, 0:_HM] = a_m
    coefs_ref[3:4, 0:_HM] = mbe1_ref[...] - a_m * mean_m


def _d_body(hgr_ref, bcat_ref, bm_ref, idx_ref, x_ref, xyz_ref, coefs_ref,
            it48_ref, wbf_ref, wbx_ref, itile_ref, fb2t_ref, xb2t_ref,
            mw2_ref, mb2_ref, sel_ref,
            out_ref):
    nb = x_ref.shape[0]
    m = out_ref.shape[0]
    a288 = coefs_ref[0:1, :]
    c288 = coefs_ref[1:2, :]
    a_m = coefs_ref[2:3, 0:_HM]
    c_m = coefs_ref[3:4, 0:_HM]

    hm = jnp.maximum(a_m * bm_ref[...] + c_m, 0.0)
    p = jnp.dot(hm, mw2_ref[...].T,
                preferred_element_type=jnp.float32) + mb2_ref[...]
    pp = jnp.sum(p * p, axis=1, keepdims=True)
    pn = jnp.maximum(jnp.sqrt(pp), 1e-12)
    prep6 = jnp.dot(p, itile_ref[...], preferred_element_type=jnp.float32)

    bcatt = jnp.dot(bcat_ref[...], it48_ref[...],
                    preferred_element_type=jnp.float32)
    ht = jnp.maximum((hgr_ref[...] - bcatt) * a288 + c288, 0.0)
    wf = jnp.dot(ht, wbf_ref[...],
                 preferred_element_type=jnp.float32) + fb2t_ref[...]
    wx = jnp.dot(ht, wbx_ref[...],
                 preferred_element_type=jnp.float32) + xb2t_ref[...]

    sel = sel_ref[...]
    u = jnp.dot(wf * prep6, sel, preferred_element_type=jnp.float32)
    v = jnp.dot(wx * prep6, sel, preferred_element_type=jnp.float32)
    s2 = jnp.dot(wf * wf, sel, preferred_element_type=jnp.float32)
    t2 = jnp.dot(wx * wx, sel, preferred_element_type=jnp.float32)

    df = jnp.maximum(jnp.sqrt(s2[:, 0:_K]), 1e-12)
    dx = jnp.maximum(jnp.sqrt(t2[:, 0:_K]), 1e-12)
    logits = (u[:, 0:_K] * v[:, 0:_K]) / (df * dx * (pn * pn))

    mx = jnp.max(logits, axis=1, keepdims=True)
    e = jnp.exp(logits - mx)
    biw = e / jnp.sum(e, axis=1, keepdims=True)

    lane = jax.lax.broadcasted_iota(jnp.int32, (nb, m), 1)
    s = jnp.zeros((nb, m), jnp.float32)
    for k in range(_K):
        s = s + biw[:, k:k + 1] * (idx_ref[:, k:k + 1] == lane).astype(jnp.float32)

    x136 = jnp.concatenate(
        [x_ref[...], xyz_ref[...], jnp.ones((nb, 1), jnp.float32),
         jnp.zeros((nb, 4), jnp.float32)], axis=1)

    dn = (((0,), (0,)), ((), ()))
    upd = jax.lax.dot_general(s, x136, dn, preferred_element_type=jnp.float32)

    @pl.when(pl.program_id(0) == 0)
    def _():
        out_ref[...] = jnp.zeros_like(out_ref)

    out_ref[...] = out_ref[...] + upd

    @pl.when(pl.program_id(0) == pl.num_programs(0) - 1)
    def _():
        den = out_ref[:, 131:132] + 1e-8
        out_ref[...] = out_ref[...] / den


def kernel(sp_fea, sp_xyz, o_p_fea, p_xyz, c2p_idx_abs, c2p_idx, cluster_idx,
           offset, sp_offset,
           fea_w1, fea_b1, fea_g1, fea_be1, fea_w2, fea_b2,
           xyz_w1, xyz_b1, xyz_g1, xyz_be1, xyz_w2, xyz_b2,
           mlp_w1, mlp_b1, mlp_g1, mlp_be1, mlp_w2, mlp_b2):
    n, c = o_p_fea.shape
    m = sp_fea.shape[0]
    f32 = jnp.float32

    nb = n
    for cand in (1000, 500, 250, 200, 100, 50, 25, 10, 8, 5, 4, 2, 1):
        if n % cand == 0:
            nb = cand
            break
    grid = n // nb

    r = lambda v: v.reshape(1, -1)

    zf = jnp.zeros((_GWK, _H2K), f32)
    zx = jnp.zeros((_GWK, _H2K), f32)
    it = jnp.zeros((_H2, _H2K), f32)
    it48 = jnp.zeros((_GW, _GWK), f32)
    sel = jnp.zeros((_H2K, 8), f32)
    eye16 = jnp.eye(_H2, dtype=f32)
    eye48 = jnp.eye(_GW, dtype=f32)
    for k in range(_K):
        zf = zf.at[k * _GW:k * _GW + _HF, k * _H2:(k + 1) * _H2].set(fea_w2.T)
        zx = zx.at[k * _GW + _HF:(k + 1) * _GW, k * _H2:(k + 1) * _H2].set(xyz_w2.T)
        it = it.at[:, k * _H2:(k + 1) * _H2].set(eye16)
        it48 = it48.at[:, k * _GW:(k + 1) * _GW].set(eye48)
        sel = sel.at[k * _H2:(k + 1) * _H2, k].set(1.0)
    fb2t = jnp.tile(fea_b2, _K).reshape(1, _H2K)
    xb2t = jnp.tile(xyz_b2, _K).reshape(1, _H2K)

    gcat = pl.pallas_call(
        _prep_body,
        out_shape=jax.ShapeDtypeStruct((m, _GW), f32),
    )(sp_fea, sp_xyz, fea_w1, r(fea_b1), xyz_w1, r(xyz_b1))

    total = n * _K
    chunk = 480
    n_chunks = -(-(-(-total // _NW)) // chunk)
    rows_w = n_chunks * chunk
    total_pad = rows_w * _NW
    idx_flat = c2p_idx_abs.reshape(-1)
    if total_pad > total:
        idx_flat = jnp.concatenate(
            [idx_flat, jnp.zeros((total_pad - total,), jnp.int32)])

    sc_gather = functools.partial(
        pl.kernel,
        out_type=jax.ShapeDtypeStruct((total_pad, _GW), f32),
        mesh=plsc.VectorSubcoreMesh(core_axis_name="c", subcore_axis_name="s"),
        scratch_types=(
            [pltpu.VMEM((chunk,), jnp.int32)] * _NBUF
            + [pltpu.VMEM((chunk, _GW), f32)] * _NBUF
            + [pltpu.SemaphoreType.DMA] * (2 * _NBUF)
        ),
        compiler_params=pltpu.CompilerParams(use_tc_tiling_on_sc=False),
    )(functools.partial(_sc_gather_body, n_chunks, chunk, rows_w))
    hgr = sc_gather(gcat, idx_flat).reshape(total_pad // _K, _GWK)

    blk = lambda shape: pl.BlockSpec(shape, lambda i: (i, 0))
    full = lambda shape: pl.BlockSpec(shape, lambda i: (0, 0))

    bcat, bm, statsm = pl.pallas_call(
        _b_body,
        grid=(grid,),
        in_specs=[
            blk((nb, c)), blk((nb, 3)),
            full((_HF, c)), full((_HM, c)), full((1, _HM)), full((_HX, 3)),
        ],
        out_specs=[blk((nb, _GW)), blk((nb, _HM)), full((8, 128))],
        out_shape=[
            jax.ShapeDtypeStruct((n, _GW), f32),
            jax.ShapeDtypeStruct((n, _HM), f32),
            jax.ShapeDtypeStruct((8, 128), f32),
        ],
    )(o_p_fea, p_xyz, fea_w1, mlp_w1, r(mlp_b1), xyz_w1)

    stats = pl.pallas_call(
        _s_body,
        grid=(grid,),
        in_specs=[blk((nb, _GWK)), blk((nb, _GW))],
        out_specs=[full((8, 128))],
        out_shape=[jax.ShapeDtypeStruct((8, 128), f32)],
    )(hgr, bcat)[0]

    coefs = pl.pallas_call(
        functools.partial(_c_body, n),
        out_shape=jax.ShapeDtypeStruct((8, _GWK), f32),
    )(stats, statsm, r(fea_g1), r(fea_be1), r(xyz_g1), r(xyz_be1),
      r(mlp_g1), r(mlp_be1))

    out = pl.pallas_call(
        _d_body,
        grid=(grid,),
        in_specs=[
            blk((nb, _GWK)), blk((nb, _GW)), blk((nb, _HM)), blk((nb, _K)),
            blk((nb, c)), blk((nb, 3)),
            full((8, _GWK)), full((_GW, _GWK)),
            full((_GWK, _H2K)), full((_GWK, _H2K)), full((_H2, _H2K)),
            full((1, _H2K)), full((1, _H2K)),
            full((_H2, _HM)), full((1, _H2)), full((_H2K, 8)),
        ],
        out_specs=[full((m, 136))],
        out_shape=[jax.ShapeDtypeStruct((m, 136), f32)],
    )(hgr, bcat, bm, c2p_idx, o_p_fea, p_xyz, coefs,
      it48, zf, zx, it, fb2t, xb2t, mlp_w2, r(mlp_b2), sel)

    out = out[0]
    return (out[:, :c], out[:, c:c + 3])
